# Initial kernel scaffold; baseline (speedup 1.0000x reference)
import jax
import jax.numpy as jnp
from jax.experimental import pallas as pl


def kernel(x, pos_edge_index, neg_edge_index, W, b, gamma, beta):
    raise NotImplementedError("write your pallas kernel here")


# speedup vs baseline: 6.7873x; 6.7873x over previous
import jax
import jax.numpy as jnp
from jax import lax
from jax.experimental import pallas as pl
from jax.experimental.pallas import tpu as pltpu

N = 10000
D = 128
E = 160000
BLK = 2000


def _tc_a_body(x_ref, w_ref, cnt_ref, ys_ref, dinv_ref):
    deg = cnt_ref[...] + 1.0
    dinv = lax.rsqrt(deg)
    dinv_ref[...] = dinv
    xw = jnp.dot(x_ref[...], w_ref[...], preferred_element_type=jnp.float32)
    ys_ref[...] = xw * dinv


_tc_a = pl.pallas_call(
    _tc_a_body,
    grid=(N // BLK,),
    in_specs=[
        pl.BlockSpec((BLK, D), lambda i: (i, 0)),
        pl.BlockSpec((D, D), lambda i: (0, 0)),
        pl.BlockSpec((BLK, 1), lambda i: (i, 0)),
    ],
    out_specs=[
        pl.BlockSpec((BLK, D), lambda i: (i, 0)),
        pl.BlockSpec((BLK, 1), lambda i: (i, 0)),
    ],
    out_shape=[
        jax.ShapeDtypeStruct((N, D), jnp.float32),
        jax.ShapeDtypeStruct((N, 1), jnp.float32),
    ],
)


def _tc_b_body(acc_ref, ys_ref, dinv_ref, b_ref, g_ref, bt_ref,
               out_ref, sum_ref, sq_ref):
    p = pl.program_id(0)
    t = (acc_ref[...] + ys_ref[...]) * dinv_ref[...] + b_ref[...]

    @pl.when(p == 0)
    def _():
        i = pl.program_id(1)

        @pl.when(i == 0)
        def _():
            sum_ref[...] = jnp.zeros_like(sum_ref)
            sq_ref[...] = jnp.zeros_like(sq_ref)

        sum_ref[...] += jnp.sum(t, axis=0, keepdims=True)
        sq_ref[...] += jnp.sum(t * t, axis=0, keepdims=True)

    @pl.when(p == 1)
    def _():
        mean = sum_ref[...] * (1.0 / N)
        var = sq_ref[...] * (1.0 / N) - mean * mean
        out_ref[...] = jnp.maximum(
            g_ref[...] * (t - mean) * lax.rsqrt(var + 1e-5) + bt_ref[...],
            0.0)


_tc_b = pl.pallas_call(
    _tc_b_body,
    grid=(2, N // BLK),
    in_specs=[
        pl.BlockSpec((BLK, D), lambda p, i: (i, 0)),
        pl.BlockSpec((BLK, D), lambda p, i: (i, 0)),
        pl.BlockSpec((BLK, 1), lambda p, i: (i, 0)),
        pl.BlockSpec((1, D), lambda p, i: (0, 0)),
        pl.BlockSpec((1, D), lambda p, i: (0, 0)),
        pl.BlockSpec((1, D), lambda p, i: (0, 0)),
    ],
    out_specs=pl.BlockSpec((BLK, D), lambda p, i: (i, 0)),
    out_shape=jax.ShapeDtypeStruct((N, D), jnp.float32),
    scratch_shapes=[
        pltpu.VMEM((1, D), jnp.float32),
        pltpu.VMEM((1, D), jnp.float32),
    ],
)


@jax.jit
def kernel(x, pos_edge_index, neg_edge_index, W, b, gamma, beta):
    del neg_edge_index
    row = pos_edge_index[0].astype(jnp.int32)
    col = pos_edge_index[1].astype(jnp.int32)
    cnt = jax.ops.segment_sum(jnp.ones((E,), jnp.float32), col,
                              num_segments=N)
    ys, dinv = _tc_a(x, W, cnt[:, None])
    acc = jax.ops.segment_sum(jnp.take(ys, row, axis=0), col,
                              num_segments=N)
    return _tc_b(acc, ys, dinv,
                 b.reshape(1, D), gamma.reshape(1, D), beta.reshape(1, D))
